# baseline (speedup 1.0000x reference)
import functools
import jax
import jax.numpy as jnp
from jax import lax
from jax.experimental import pallas as pl
from jax.experimental.pallas import tpu as pltpu
from jax.experimental.pallas import tpu_sc as plsc

_D = 256
_I = 64
_J = 128
_GRP = 16


def _sc_body(s, n_emb, emb_hbm, out_hbm, tbl_v, idx_v, sem):
    n_rows = _I + _J
    wid = lax.axis_index("s") * 2 + lax.axis_index("c")
    it = wid // (s // _J)
    jt = wid % (s // _J)
    i0 = it * _I
    j0 = jt * _J
    r0 = pl.multiple_of(i0 - j0 + (s - 1) - (_J - 1), 8)

    pltpu.sync_copy(emb_hbm.at[pl.ds(r0, n_rows)], tbl_v)

    lane = lax.broadcasted_iota(jnp.int32, (16,), 0)

    def build_row(li, _):
        a = (i0 + li) * s + j0 + (_J - 1)
        for c in range(_J // 16):
            idx_v[li, pl.ds(c * 16, 16)] = (a - c * 16) - lane
        return _

    lax.fori_loop(0, _I, build_row, None)

    def scatter_group(g, _):
        base = g * _GRP
        for b in range(_GRP):
            pltpu.make_async_copy(
                tbl_v.at[pl.ds(base + b, _J)],
                out_hbm.at[idx_v.at[base + b]],
                sem,
            ).start()
        for b in range(_GRP):
            pltpu.make_async_copy(
                tbl_v.at[pl.ds(base + b, _J)],
                out_hbm.at[idx_v.at[base + b]],
                sem,
            ).wait()
        return _

    lax.fori_loop(0, _I // _GRP, scatter_group, None)


def kernel(seq_len, rel_pos_emb):
    n_emb, d = rel_pos_emb.shape
    s = (n_emb + 1) // 2

    mesh = plsc.VectorSubcoreMesh(core_axis_name="c", subcore_axis_name="s")
    body = functools.partial(_sc_body, s, n_emb)
    sc_kernel = pl.kernel(
        body,
        mesh=mesh,
        out_type=jax.ShapeDtypeStruct((s * s, d), rel_pos_emb.dtype),
        scratch_types=[
            pltpu.VMEM((_I + _J, d), rel_pos_emb.dtype),
            pltpu.VMEM((_I, _J), jnp.int32),
            pltpu.SemaphoreType.DMA,
        ],
        compiler_params=pltpu.CompilerParams(use_tc_tiling_on_sc=False),
    )
    emb_pad = jnp.concatenate(
        [rel_pos_emb, jnp.zeros((1, d), rel_pos_emb.dtype)], axis=0)
    out = sc_kernel(emb_pad)
    return out.reshape(s, s, d)


# speedup vs baseline: 1.0029x; 1.0029x over previous
import functools
import jax
import jax.numpy as jnp
from jax import lax
from jax.experimental import pallas as pl
from jax.experimental.pallas import tpu as pltpu
from jax.experimental.pallas import tpu_sc as plsc

_D = 256
_I = 64
_J = 128
_GRP = 16


def _sc_body(s, n_emb, emb_hbm, out_hbm, tbl_v, idx_v, sem):
    n_rows = _I + _J
    wid = lax.axis_index("s") * 2 + lax.axis_index("c")
    it = wid // (s // _J)
    jt = wid % (s // _J)
    i0 = it * _I
    j0 = jt * _J
    r0 = pl.multiple_of(i0 - j0 + (s - 1) - (_J - 1), 8)

    pltpu.sync_copy(emb_hbm.at[pl.ds(r0, n_rows)], tbl_v)

    lane = lax.broadcasted_iota(jnp.int32, (16,), 0)

    def build_row(li, _):
        a = (i0 + li) * s + j0 + (_J - 1)
        for c in range(_J // 16):
            idx_v[li, pl.ds(c * 16, 16)] = (a - c * 16) - lane
        return _

    lax.fori_loop(0, _I, build_row, None)

    def fire_group(g, _):
        base = g * _GRP
        for b in range(_GRP):
            pltpu.make_async_copy(
                tbl_v.at[pl.ds(base + b, _J)],
                out_hbm.at[idx_v.at[base + b]],
                sem,
            ).start()
        return _

    def drain_group(g, _):
        base = g * _GRP
        for b in range(_GRP):
            pltpu.make_async_copy(
                tbl_v.at[pl.ds(base + b, _J)],
                out_hbm.at[idx_v.at[base + b]],
                sem,
            ).wait()
        return _

    lax.fori_loop(0, _I // _GRP, fire_group, None)
    lax.fori_loop(0, _I // _GRP, drain_group, None)


def kernel(seq_len, rel_pos_emb):
    n_emb, d = rel_pos_emb.shape
    s = (n_emb + 1) // 2

    mesh = plsc.VectorSubcoreMesh(core_axis_name="c", subcore_axis_name="s")
    body = functools.partial(_sc_body, s, n_emb)
    sc_kernel = pl.kernel(
        body,
        mesh=mesh,
        out_type=jax.ShapeDtypeStruct((s * s, d), rel_pos_emb.dtype),
        scratch_types=[
            pltpu.VMEM((_I + _J, d), rel_pos_emb.dtype),
            pltpu.VMEM((_I, _J), jnp.int32),
            pltpu.SemaphoreType.DMA,
        ],
        compiler_params=pltpu.CompilerParams(use_tc_tiling_on_sc=False),
    )
    emb_pad = jnp.concatenate(
        [rel_pos_emb, jnp.zeros((1, d), rel_pos_emb.dtype)], axis=0)
    out = sc_kernel(emb_pad)
    return out.reshape(s, s, d)
